# Initial kernel scaffold; baseline (speedup 1.0000x reference)
import jax
import jax.numpy as jnp
from jax.experimental import pallas as pl


def kernel(features, indices, gamma1, beta1, mean1, var1, W1, gamma2, beta2, mean2, var2, W2, Wse1, bse1, Wse2, bse2):
    raise NotImplementedError("write your pallas kernel here")


# speedup vs baseline: 19.1978x; 19.1978x over previous
import functools

import jax
import jax.numpy as jnp
from jax import lax
from jax.experimental import pallas as pl
from jax.experimental.pallas import tpu as pltpu
from jax.experimental.pallas import tpu_sc as plsc

BB, HH, WW = 16, 512, 512
EPS = 1e-5
LANES = 16
NWORK = 32
CHUNK = 256
NIDX = 128
BLK = 512

_SC_PARAMS = pltpu.CompilerParams(use_tc_tiling_on_sc=False)


def _dense_taps_body(nrows, f_ref, sc_ref, sh_ref, w_ref, *out_refs):
    i = pl.program_id(0)
    c = f_ref.shape[1]
    rows = i * BLK + lax.broadcasted_iota(jnp.int32, (BLK, 1), 0)
    z = jnp.maximum(f_ref[...] * sc_ref[...] + sh_ref[...], 0.0)
    r = jnp.dot(z, w_ref[...], preferred_element_type=jnp.float32)
    r = jnp.where(rows < nrows, r, 0.0)
    for k in range(9):
        out_refs[k][...] = r[:, k * c:(k + 1) * c]


def _dense_taps(fpad, scale, shift, wcat, nrows):
    npad, c = fpad.shape
    nblk = npad // BLK
    outs = pl.pallas_call(
        functools.partial(_dense_taps_body, nrows),
        grid=(nblk,),
        in_specs=[
            pl.BlockSpec((BLK, c), lambda i: (i, 0)),
            pl.BlockSpec((1, c), lambda i: (0, 0)),
            pl.BlockSpec((1, c), lambda i: (0, 0)),
            pl.BlockSpec((c, 9 * c), lambda i: (0, 0)),
        ],
        out_specs=[pl.BlockSpec((BLK, c), lambda i: (i, 0))] * 9,
        out_shape=[jax.ShapeDtypeStruct((npad, c), jnp.float32)] * 9,
    )(fpad, scale, shift, wcat)
    return outs


def _sc_mesh():
    return plsc.VectorSubcoreMesh(core_axis_name="c", subcore_axis_name="s",
                                  num_cores=2, num_subcores=16)


def _sc_conv_a(grid_flat, base_flat, ys, npad, c, sent):
    offs = [dy * (WW + 2) + dx for dy in (-1, 0, 1) for dx in (-1, 0, 1)]
    per = npad // NWORK
    nsub = per // CHUNK
    nd = CHUNK // NIDX
    nc = 2

    @functools.partial(
        pl.kernel, mesh=_sc_mesh(), compiler_params=_SC_PARAMS,
        out_type=[jax.ShapeDtypeStruct((npad, c), jnp.float32)]
        + [jax.ShapeDtypeStruct((npad,), jnp.int32)] * 9,
        scratch_types=(
            [pltpu.VMEM((CHUNK,), jnp.int32)]
            + [pltpu.VMEM((CHUNK,), jnp.int32) for _ in range(9)]
            + [pltpu.VMEM((CHUNK,), jnp.int32) for _ in range(9)]
            + [pltpu.VMEM((CHUNK,), jnp.int32) for _ in range(9)]
            + [pltpu.VMEM((CHUNK, c), jnp.float32) for _ in range(9)]
            + [pltpu.VMEM((CHUNK, c), jnp.float32)]
            + [pltpu.SemaphoreType.DMA, pltpu.SemaphoreType.DMA]
        ),
    )
    def body(grid_hbm, base_hbm, y0, y1, y2, y3, y4, y5, y6, y7, y8,
             h_out, i0, i1, i2, i3, i4, i5, i6, i7, i8,
             base_v,
             g0, g1, g2, g3, g4, g5, g6, g7, g8,
             n0, n1, n2, n3, n4, n5, n6, n7, n8,
             x0, x1, x2, x3, x4, x5, x6, x7, x8,
             b0, b1, b2, b3, b4, b5, b6, b7, b8,
             acc_v, sem_g, sem_y):
        yrefs = (y0, y1, y2, y3, y4, y5, y6, y7, y8)
        irefs = (i0, i1, i2, i3, i4, i5, i6, i7, i8)
        grefs = (g0, g1, g2, g3, g4, g5, g6, g7, g8)
        nrefs = (n0, n1, n2, n3, n4, n5, n6, n7, n8)
        xrefs = (x0, x1, x2, x3, x4, x5, x6, x7, x8)
        brefs = (b0, b1, b2, b3, b4, b5, b6, b7, b8)
        wid = lax.axis_index("s") * nc + lax.axis_index("c")

        def step(t, _):
            gb = wid * per + t * CHUNK
            pltpu.sync_copy(base_hbm.at[pl.ds(gb, CHUNK)], base_v)

            def off_lane(j, _):
                bv = base_v[pl.ds(j * LANES, LANES)]
                for k in range(9):
                    grefs[k][pl.ds(j * LANES, LANES)] = bv + offs[k]
                return 0
            lax.fori_loop(0, CHUNK // LANES, off_lane, 0)

            gh = []
            for k in range(9):
                for d in range(nd):
                    gh.append(pltpu.async_copy(
                        grid_hbm.at[grefs[k].at[pl.ds(d * NIDX, NIDX)]],
                        nrefs[k].at[pl.ds(d * NIDX, NIDX)], sem_g))
            for h in gh:
                h.wait()

            def fix_lane(j, _):
                for k in range(9):
                    nb = nrefs[k][pl.ds(j * LANES, LANES)]
                    xrefs[k][pl.ds(j * LANES, LANES)] = jnp.where(
                        nb < 0, sent, nb)
                return 0
            lax.fori_loop(0, CHUNK // LANES, fix_lane, 0)

            yh = []
            for k in range(9):
                for d in range(nd):
                    yh.append(pltpu.async_copy(
                        yrefs[k].at[xrefs[k].at[pl.ds(d * NIDX, NIDX)]],
                        brefs[k].at[pl.ds(d * NIDX, NIDX)], sem_y))
                pltpu.sync_copy(xrefs[k], irefs[k].at[pl.ds(gb, CHUNK)])
            for h in yh:
                h.wait()

            def red(r, _):
                for hh in range(c // LANES):
                    s = brefs[0][r, pl.ds(hh * LANES, LANES)]
                    for k in range(1, 9):
                        s = s + brefs[k][r, pl.ds(hh * LANES, LANES)]
                    acc_v[r, pl.ds(hh * LANES, LANES)] = s
                return 0
            lax.fori_loop(0, CHUNK, red, 0)
            pltpu.sync_copy(acc_v, h_out.at[pl.ds(gb, CHUNK)])
            return 0

        lax.fori_loop(0, nsub, step, 0)

    return body(grid_flat, base_flat, *ys)


def _sc_conv_b(idxs, ys, npad, c):
    per = npad // NWORK
    nsub = per // CHUNK
    nd = CHUNK // NIDX
    nc = 2

    @functools.partial(
        pl.kernel, mesh=_sc_mesh(), compiler_params=_SC_PARAMS,
        out_type=jax.ShapeDtypeStruct((npad, c), jnp.float32),
        scratch_types=(
            [pltpu.VMEM((CHUNK,), jnp.int32) for _ in range(9)]
            + [pltpu.VMEM((CHUNK, c), jnp.float32) for _ in range(9)]
            + [pltpu.VMEM((CHUNK, c), jnp.float32)]
            + [pltpu.SemaphoreType.DMA]
        ),
    )
    def body(i0, i1, i2, i3, i4, i5, i6, i7, i8,
             y0, y1, y2, y3, y4, y5, y6, y7, y8, h_out,
             x0, x1, x2, x3, x4, x5, x6, x7, x8,
             b0, b1, b2, b3, b4, b5, b6, b7, b8,
             acc_v, sem_y):
        irefs = (i0, i1, i2, i3, i4, i5, i6, i7, i8)
        yrefs = (y0, y1, y2, y3, y4, y5, y6, y7, y8)
        xrefs = (x0, x1, x2, x3, x4, x5, x6, x7, x8)
        brefs = (b0, b1, b2, b3, b4, b5, b6, b7, b8)
        wid = lax.axis_index("s") * nc + lax.axis_index("c")

        def step(t, _):
            gb = wid * per + t * CHUNK
            for k in range(9):
                pltpu.sync_copy(irefs[k].at[pl.ds(gb, CHUNK)], xrefs[k])
            yh = []
            for k in range(9):
                for d in range(nd):
                    yh.append(pltpu.async_copy(
                        yrefs[k].at[xrefs[k].at[pl.ds(d * NIDX, NIDX)]],
                        brefs[k].at[pl.ds(d * NIDX, NIDX)], sem_y))
            for h in yh:
                h.wait()

            def red(r, _):
                for hh in range(c // LANES):
                    s = brefs[0][r, pl.ds(hh * LANES, LANES)]
                    for k in range(1, 9):
                        s = s + brefs[k][r, pl.ds(hh * LANES, LANES)]
                    acc_v[r, pl.ds(hh * LANES, LANES)] = s
                return 0
            lax.fori_loop(0, CHUNK, red, 0)
            pltpu.sync_copy(acc_v, h_out.at[pl.ds(gb, CHUNK)])
            return 0

        lax.fori_loop(0, nsub, step, 0)

    return body(*idxs, *ys)


def _pool_body(h_ref, b_ref, pooled_ref, cnt_ref):
    i = pl.program_id(0)
    bvec = b_ref[0, 0, :]
    onehot = (bvec[None, :] == lax.broadcasted_iota(
        jnp.int32, (BB, BLK), 0)).astype(jnp.float32)

    @pl.when(i == 0)
    def _():
        pooled_ref[...] = jnp.zeros_like(pooled_ref)
        cnt_ref[...] = jnp.zeros_like(cnt_ref)

    pooled_ref[...] += jnp.dot(onehot, h_ref[...],
                               preferred_element_type=jnp.float32)
    cnt_ref[...] += jnp.broadcast_to(
        jnp.sum(onehot, axis=1)[:, None], cnt_ref.shape)


def _pool(h2, bidx3, c):
    npad = h2.shape[0]
    nblk = npad // BLK
    return pl.pallas_call(
        _pool_body,
        grid=(nblk,),
        in_specs=[
            pl.BlockSpec((BLK, c), lambda i: (i, 0)),
            pl.BlockSpec((1, 1, BLK), lambda i: (i, 0, 0)),
        ],
        out_specs=[pl.BlockSpec((BB, c), lambda i: (0, 0)),
                   pl.BlockSpec((BB, 128), lambda i: (0, 0))],
        out_shape=[jax.ShapeDtypeStruct((BB, c), jnp.float32),
                   jax.ShapeDtypeStruct((BB, 128), jnp.float32)],
    )(h2, bidx3)


def _finish_body(h_ref, f_ref, b_ref, se_ref, out_ref):
    bvec = b_ref[0, 0, :]
    onehot = (bvec[:, None] == lax.broadcasted_iota(
        jnp.int32, (BLK, BB), 1)).astype(jnp.float32)
    se_rows = jnp.dot(onehot, se_ref[...], preferred_element_type=jnp.float32)
    out_ref[...] = h_ref[...] * se_rows + f_ref[...]


def _finish(h2, fpad, bidx3, se, c):
    npad = h2.shape[0]
    nblk = npad // BLK
    return pl.pallas_call(
        _finish_body,
        grid=(nblk,),
        in_specs=[
            pl.BlockSpec((BLK, c), lambda i: (i, 0)),
            pl.BlockSpec((BLK, c), lambda i: (i, 0)),
            pl.BlockSpec((1, 1, BLK), lambda i: (i, 0, 0)),
            pl.BlockSpec((BB, c), lambda i: (0, 0)),
        ],
        out_specs=pl.BlockSpec((BLK, c), lambda i: (i, 0)),
        out_shape=jax.ShapeDtypeStruct((npad, c), jnp.float32),
    )(h2, fpad, bidx3, se)


def kernel(features, indices, gamma1, beta1, mean1, var1, W1,
           gamma2, beta2, mean2, var2, W2, Wse1, bse1, Wse2, bse2):
    n, c = features.shape
    sent = n
    seg = NWORK * CHUNK
    npad = ((n + 1 + seg - 1) // seg) * seg
    nblk = npad // BLK

    bidx = indices[:, 0].astype(jnp.int32)
    yy = indices[:, 1].astype(jnp.int32) + 1
    xx = indices[:, 2].astype(jnp.int32) + 1
    base_flat = (bidx * (HH + 2) + yy) * (WW + 2) + xx
    gsize = BB * (HH + 2) * (WW + 2)
    grid_flat = jnp.full((gsize,), -1, jnp.int32).at[base_flat].set(
        jnp.arange(n, dtype=jnp.int32))
    base_pad = jnp.full((npad,), (WW + 2) + 1, jnp.int32).at[:n].set(base_flat)
    bidx_pad = jnp.full((npad,), BB, jnp.int32).at[:n].set(bidx)
    bidx3 = bidx_pad.reshape(nblk, 1, BLK)
    fpad = jnp.zeros((npad, c), jnp.float32).at[:n].set(features)

    def bn_consts(g, b, m, v):
        s = g / jnp.sqrt(v + EPS)
        return (s.reshape(1, c), (b - m * s).reshape(1, c))

    sc1, sh1 = bn_consts(gamma1, beta1, mean1, var1)
    sc2, sh2 = bn_consts(gamma2, beta2, mean2, var2)
    w1cat = jnp.transpose(W1, (1, 0, 2)).reshape(c, 9 * c)
    w2cat = jnp.transpose(W2, (1, 0, 2)).reshape(c, 9 * c)

    ys1 = _dense_taps(fpad, sc1, sh1, w1cat, n)
    res1 = _sc_conv_a(grid_flat, base_pad, ys1, npad, c, sent)
    h1, idxs = res1[0], res1[1:]
    ys2 = _dense_taps(h1, sc2, sh2, w2cat, n)
    h2 = _sc_conv_b(idxs, ys2, npad, c)

    pooled, cnt = _pool(h2, bidx3, c)
    counts = jnp.maximum(cnt[:, 0], 1.0)
    pooled = pooled / counts[:, None]
    se = jax.nn.sigmoid(
        jax.nn.relu(pooled @ Wse1 + bse1) @ Wse2 + bse2)

    out = _finish(h2, fpad, bidx3, se, c)
    return out[:n]


# speedup vs baseline: 19.2590x; 1.0032x over previous
import functools

import jax
import jax.numpy as jnp
from jax import lax
from jax.experimental import pallas as pl
from jax.experimental.pallas import tpu as pltpu
from jax.experimental.pallas import tpu_sc as plsc

BB, HH, WW = 16, 512, 512
EPS = 1e-5
LANES = 16
NWORK = 32
CHUNK = 256
NIDX = 128
BLK = 512

_SC_PARAMS = pltpu.CompilerParams(use_tc_tiling_on_sc=False)


def _dense_taps_body(nrows, f_ref, sc_ref, sh_ref, w_ref, *out_refs):
    i = pl.program_id(0)
    c = f_ref.shape[1]
    rows = i * BLK + lax.broadcasted_iota(jnp.int32, (BLK, 1), 0)
    z = jnp.maximum(f_ref[...] * sc_ref[...] + sh_ref[...], 0.0)
    r = jnp.dot(z, w_ref[...], preferred_element_type=jnp.float32)
    r = jnp.where(rows < nrows, r, 0.0)
    for k in range(9):
        out_refs[k][...] = r[:, k * c:(k + 1) * c]


def _dense_taps(fpad, scale, shift, wcat, nrows):
    npad, c = fpad.shape
    nblk = npad // BLK
    outs = pl.pallas_call(
        functools.partial(_dense_taps_body, nrows),
        grid=(nblk,),
        in_specs=[
            pl.BlockSpec((BLK, c), lambda i: (i, 0)),
            pl.BlockSpec((1, c), lambda i: (0, 0)),
            pl.BlockSpec((1, c), lambda i: (0, 0)),
            pl.BlockSpec((c, 9 * c), lambda i: (0, 0)),
        ],
        out_specs=[pl.BlockSpec((BLK, c), lambda i: (i, 0))] * 9,
        out_shape=[jax.ShapeDtypeStruct((npad, c), jnp.float32)] * 9,
    )(fpad, scale, shift, wcat)
    return outs


def _sc_mesh():
    return plsc.VectorSubcoreMesh(core_axis_name="c", subcore_axis_name="s",
                                  num_cores=2, num_subcores=16)


def _sc_conv_a(grid_flat, base_flat, ys, npad, c, sent):
    offs = [dy * (WW + 2) + dx for dy in (-1, 0, 1) for dx in (-1, 0, 1)]
    per = npad // NWORK
    nsub = per // CHUNK
    nd = CHUNK // NIDX
    nc = 2

    @functools.partial(
        pl.kernel, mesh=_sc_mesh(), compiler_params=_SC_PARAMS,
        out_type=[jax.ShapeDtypeStruct((npad, c), jnp.float32)]
        + [jax.ShapeDtypeStruct((npad,), jnp.int32)] * 8,
        scratch_types=(
            [pltpu.VMEM((CHUNK,), jnp.int32)]
            + [pltpu.VMEM((CHUNK,), jnp.int32) for _ in range(9)]
            + [pltpu.VMEM((CHUNK,), jnp.int32) for _ in range(9)]
            + [pltpu.VMEM((CHUNK,), jnp.int32) for _ in range(9)]
            + [pltpu.VMEM((CHUNK, c), jnp.float32) for _ in range(9)]
            + [pltpu.VMEM((CHUNK, c), jnp.float32)]
            + [pltpu.SemaphoreType.DMA, pltpu.SemaphoreType.DMA]
        ),
    )
    def body(grid_hbm, base_hbm, y0, y1, y2, y3, y4, y5, y6, y7, y8,
             h_out, i0, i1, i2, i3, i4, i5, i6, i7,
             base_v,
             g0, g1, g2, g3, g4, g5, g6, g7, g8,
             n0, n1, n2, n3, n4, n5, n6, n7, n8,
             x0, x1, x2, x3, x4, x5, x6, x7, x8,
             b0, b1, b2, b3, b4, b5, b6, b7, b8,
             acc_v, sem_g, sem_y):
        yrefs = (y0, y1, y2, y3, y4, y5, y6, y7, y8)
        irefs = (i0, i1, i2, i3, i4, i5, i6, i7)
        grefs = (g0, g1, g2, g3, g4, g5, g6, g7, g8)
        taps = (0, 1, 2, 3, 5, 6, 7, 8)
        nrefs = (n0, n1, n2, n3, n4, n5, n6, n7, n8)
        xrefs = (x0, x1, x2, x3, x4, x5, x6, x7, x8)
        brefs = (b0, b1, b2, b3, b4, b5, b6, b7, b8)
        wid = lax.axis_index("s") * nc + lax.axis_index("c")

        def step(t, _):
            gb = wid * per + t * CHUNK
            pltpu.sync_copy(base_hbm.at[pl.ds(gb, CHUNK)], base_v)

            def off_lane(j, _):
                bv = base_v[pl.ds(j * LANES, LANES)]
                for k in taps:
                    grefs[k][pl.ds(j * LANES, LANES)] = bv + offs[k]
                return 0
            lax.fori_loop(0, CHUNK // LANES, off_lane, 0)

            gh = []
            for k in taps:
                for d in range(nd):
                    gh.append(pltpu.async_copy(
                        grid_hbm.at[grefs[k].at[pl.ds(d * NIDX, NIDX)]],
                        nrefs[k].at[pl.ds(d * NIDX, NIDX)], sem_g))
            for h in gh:
                h.wait()

            def fix_lane(j, _):
                for k in taps:
                    nb = nrefs[k][pl.ds(j * LANES, LANES)]
                    xrefs[k][pl.ds(j * LANES, LANES)] = jnp.where(
                        nb < 0, sent, nb)
                return 0
            lax.fori_loop(0, CHUNK // LANES, fix_lane, 0)

            yh = [pltpu.async_copy(yrefs[4].at[pl.ds(gb, CHUNK)],
                                   brefs[4], sem_y)]
            for j, k in enumerate(taps):
                for d in range(nd):
                    yh.append(pltpu.async_copy(
                        yrefs[k].at[xrefs[k].at[pl.ds(d * NIDX, NIDX)]],
                        brefs[k].at[pl.ds(d * NIDX, NIDX)], sem_y))
                pltpu.sync_copy(xrefs[k], irefs[j].at[pl.ds(gb, CHUNK)])
            for h in yh:
                h.wait()

            def red(r, _):
                for hh in range(c // LANES):
                    s = brefs[0][r, pl.ds(hh * LANES, LANES)]
                    for k in range(1, 9):
                        s = s + brefs[k][r, pl.ds(hh * LANES, LANES)]
                    acc_v[r, pl.ds(hh * LANES, LANES)] = s
                return 0
            lax.fori_loop(0, CHUNK, red, 0)
            pltpu.sync_copy(acc_v, h_out.at[pl.ds(gb, CHUNK)])
            return 0

        lax.fori_loop(0, nsub, step, 0)

    return body(grid_flat, base_flat, *ys)


def _sc_conv_b(idxs, ys, npad, c):
    per = npad // NWORK
    nsub = per // CHUNK
    nd = CHUNK // NIDX
    nc = 2

    @functools.partial(
        pl.kernel, mesh=_sc_mesh(), compiler_params=_SC_PARAMS,
        out_type=jax.ShapeDtypeStruct((npad, c), jnp.float32),
        scratch_types=(
            [pltpu.VMEM((CHUNK,), jnp.int32) for _ in range(9)]
            + [pltpu.VMEM((CHUNK, c), jnp.float32) for _ in range(9)]
            + [pltpu.VMEM((CHUNK, c), jnp.float32)]
            + [pltpu.SemaphoreType.DMA]
        ),
    )
    def body(i0, i1, i2, i3, i4, i5, i6, i7,
             y0, y1, y2, y3, y4, y5, y6, y7, y8, h_out,
             x0, x1, x2, x3, x4, x5, x6, x7, x8,
             b0, b1, b2, b3, b4, b5, b6, b7, b8,
             acc_v, sem_y):
        irefs = (i0, i1, i2, i3, i4, i5, i6, i7)
        yrefs = (y0, y1, y2, y3, y4, y5, y6, y7, y8)
        xrefs = (x0, x1, x2, x3, x4, x5, x6, x7, x8)
        brefs = (b0, b1, b2, b3, b4, b5, b6, b7, b8)
        taps = (0, 1, 2, 3, 5, 6, 7, 8)
        wid = lax.axis_index("s") * nc + lax.axis_index("c")

        def step(t, _):
            gb = wid * per + t * CHUNK
            for j, k in enumerate(taps):
                pltpu.sync_copy(irefs[j].at[pl.ds(gb, CHUNK)], xrefs[k])
            yh = [pltpu.async_copy(yrefs[4].at[pl.ds(gb, CHUNK)],
                                   brefs[4], sem_y)]
            for k in taps:
                for d in range(nd):
                    yh.append(pltpu.async_copy(
                        yrefs[k].at[xrefs[k].at[pl.ds(d * NIDX, NIDX)]],
                        brefs[k].at[pl.ds(d * NIDX, NIDX)], sem_y))
            for h in yh:
                h.wait()

            def red(r, _):
                for hh in range(c // LANES):
                    s = brefs[0][r, pl.ds(hh * LANES, LANES)]
                    for k in range(1, 9):
                        s = s + brefs[k][r, pl.ds(hh * LANES, LANES)]
                    acc_v[r, pl.ds(hh * LANES, LANES)] = s
                return 0
            lax.fori_loop(0, CHUNK, red, 0)
            pltpu.sync_copy(acc_v, h_out.at[pl.ds(gb, CHUNK)])
            return 0

        lax.fori_loop(0, nsub, step, 0)

    return body(*idxs, *ys)


def _pool_body(h_ref, b_ref, pooled_ref, cnt_ref):
    i = pl.program_id(0)
    bvec = b_ref[0, 0, :]
    onehot = (bvec[None, :] == lax.broadcasted_iota(
        jnp.int32, (BB, BLK), 0)).astype(jnp.float32)

    @pl.when(i == 0)
    def _():
        pooled_ref[...] = jnp.zeros_like(pooled_ref)
        cnt_ref[...] = jnp.zeros_like(cnt_ref)

    pooled_ref[...] += jnp.dot(onehot, h_ref[...],
                               preferred_element_type=jnp.float32)
    cnt_ref[...] += jnp.broadcast_to(
        jnp.sum(onehot, axis=1)[:, None], cnt_ref.shape)


def _pool(h2, bidx3, c):
    npad = h2.shape[0]
    nblk = npad // BLK
    return pl.pallas_call(
        _pool_body,
        grid=(nblk,),
        in_specs=[
            pl.BlockSpec((BLK, c), lambda i: (i, 0)),
            pl.BlockSpec((1, 1, BLK), lambda i: (i, 0, 0)),
        ],
        out_specs=[pl.BlockSpec((BB, c), lambda i: (0, 0)),
                   pl.BlockSpec((BB, 128), lambda i: (0, 0))],
        out_shape=[jax.ShapeDtypeStruct((BB, c), jnp.float32),
                   jax.ShapeDtypeStruct((BB, 128), jnp.float32)],
    )(h2, bidx3)


def _finish_body(h_ref, f_ref, b_ref, se_ref, out_ref):
    bvec = b_ref[0, 0, :]
    onehot = (bvec[:, None] == lax.broadcasted_iota(
        jnp.int32, (BLK, BB), 1)).astype(jnp.float32)
    se_rows = jnp.dot(onehot, se_ref[...], preferred_element_type=jnp.float32)
    out_ref[...] = h_ref[...] * se_rows + f_ref[...]


def _finish(h2, fpad, bidx3, se, c):
    npad = h2.shape[0]
    nblk = npad // BLK
    return pl.pallas_call(
        _finish_body,
        grid=(nblk,),
        in_specs=[
            pl.BlockSpec((BLK, c), lambda i: (i, 0)),
            pl.BlockSpec((BLK, c), lambda i: (i, 0)),
            pl.BlockSpec((1, 1, BLK), lambda i: (i, 0, 0)),
            pl.BlockSpec((BB, c), lambda i: (0, 0)),
        ],
        out_specs=pl.BlockSpec((BLK, c), lambda i: (i, 0)),
        out_shape=jax.ShapeDtypeStruct((npad, c), jnp.float32),
    )(h2, fpad, bidx3, se)


def kernel(features, indices, gamma1, beta1, mean1, var1, W1,
           gamma2, beta2, mean2, var2, W2, Wse1, bse1, Wse2, bse2):
    n, c = features.shape
    sent = n
    seg = NWORK * CHUNK
    npad = ((n + 1 + seg - 1) // seg) * seg
    nblk = npad // BLK

    bidx = indices[:, 0].astype(jnp.int32)
    yy = indices[:, 1].astype(jnp.int32) + 1
    xx = indices[:, 2].astype(jnp.int32) + 1
    base_flat = (bidx * (HH + 2) + yy) * (WW + 2) + xx
    gsize = BB * (HH + 2) * (WW + 2)
    grid_flat = jnp.full((gsize,), -1, jnp.int32).at[base_flat].set(
        jnp.arange(n, dtype=jnp.int32))
    base_pad = jnp.full((npad,), (WW + 2) + 1, jnp.int32).at[:n].set(base_flat)
    bidx_pad = jnp.full((npad,), BB, jnp.int32).at[:n].set(bidx)
    bidx3 = bidx_pad.reshape(nblk, 1, BLK)
    fpad = jnp.zeros((npad, c), jnp.float32).at[:n].set(features)

    def bn_consts(g, b, m, v):
        s = g / jnp.sqrt(v + EPS)
        return (s.reshape(1, c), (b - m * s).reshape(1, c))

    sc1, sh1 = bn_consts(gamma1, beta1, mean1, var1)
    sc2, sh2 = bn_consts(gamma2, beta2, mean2, var2)
    w1cat = jnp.transpose(W1, (1, 0, 2)).reshape(c, 9 * c)
    w2cat = jnp.transpose(W2, (1, 0, 2)).reshape(c, 9 * c)

    ys1 = _dense_taps(fpad, sc1, sh1, w1cat, n)
    res1 = _sc_conv_a(grid_flat, base_pad, ys1, npad, c, sent)
    h1, idxs = res1[0], res1[1:]
    ys2 = _dense_taps(h1, sc2, sh2, w2cat, n)
    h2 = _sc_conv_b(idxs, ys2, npad, c)

    pooled, cnt = _pool(h2, bidx3, c)
    counts = jnp.maximum(cnt[:, 0], 1.0)
    pooled = pooled / counts[:, None]
    se = jax.nn.sigmoid(
        jax.nn.relu(pooled @ Wse1 + bse1) @ Wse2 + bse2)

    out = _finish(h2, fpad, bidx3, se, c)
    return out[:n]
